# baseline (speedup 1.0000x reference)
import dataclasses
import functools

import jax
import jax.numpy as jnp
from jax import lax
from jax.experimental import pallas as pl
from jax.experimental.pallas import tpu as pltpu
from jax.experimental.pallas import tpu_sc as plsc

B, L, D = 16, 2048, 300
NC, NS = 2, 16
NW = NC * NS
LANES = 16
NFG = (D + LANES - 1) // LANES
DPAD = NFG * LANES
PAIRS = 2 * B

KCOLS = 1280
CC = 256
TCR = 128
NTCH = (L - KCOLS) // TCR
CVR = TCR // LANES
ACCW = DPAD + 1


def _tc_dense_sums(prem_t, hyp_t, lens_p2, lens_h2):
    def body(lp_ref, lh_ref, p_ref, h_ref, pout_ref, hout_ref):
        j = pl.program_id(0)
        ci = lax.broadcasted_iota(jnp.int32, (1, B, CC), 2) + j * CC
        mp = (ci < lp_ref[...].reshape(1, B, 1)).astype(jnp.float32)
        mh = (ci < lh_ref[...].reshape(1, B, 1)).astype(jnp.float32)
        ps = jnp.sum(p_ref[...] * mp, axis=2)
        hs = jnp.sum(h_ref[...] * mh, axis=2)

        @pl.when(j == 0)
        def _():
            pout_ref[...] = ps
            hout_ref[...] = hs

        @pl.when(j > 0)
        def _():
            pout_ref[...] += ps
            hout_ref[...] += hs

    return pl.pallas_call(
        body,
        grid=(KCOLS // CC,),
        in_specs=[
            pl.BlockSpec((1, B), lambda j: (0, 0)),
            pl.BlockSpec((1, B), lambda j: (0, 0)),
            pl.BlockSpec((D, B, CC), lambda j: (0, 0, j)),
            pl.BlockSpec((D, B, CC), lambda j: (0, 0, j)),
        ],
        out_specs=[
            pl.BlockSpec((D, B), lambda j: (0, 0)),
            pl.BlockSpec((D, B), lambda j: (0, 0)),
        ],
        out_shape=[jax.ShapeDtypeStruct((D, B), jnp.float32)] * 2,
    )(lens_p2, lens_h2, prem_t, hyp_t)


def _sc_tail_sums(prem_t, lens_p, hyp_t, lens_h):
    mesh = plsc.VectorSubcoreMesh(
        core_axis_name="c", subcore_axis_name="s",
        num_cores=NC, num_subcores=NS)
    cp = pltpu.CompilerParams()
    if "needs_layout_passes" in pltpu.CompilerParams.__dataclass_fields__:
        cp = dataclasses.replace(cp, needs_layout_passes=False)

    @functools.partial(
        pl.kernel,
        compiler_params=cp,
        out_type=jax.ShapeDtypeStruct((NW, NTCH * ACCW * LANES),
                                      jnp.float32),
        mesh=mesh,
        scratch_types=[
            pltpu.VMEM((DPAD, TCR), jnp.float32),
            pltpu.VMEM((DPAD, TCR), jnp.float32),
            pltpu.VMEM((NTCH * ACCW * LANES,), jnp.float32),
            pltpu.VMEM((LANES,), jnp.int32),
            pltpu.VMEM((LANES,), jnp.int32),
            pltpu.SemaphoreType.DMA,
            pltpu.SemaphoreType.DMA,
        ],
    )
    def k(prem_hbm, lenp_hbm, hyp_hbm, lenh_hbm, out_hbm,
          buf0, buf1, stage, lpv, lhv, sem0, sem1):
        wid = lax.axis_index("s") * NC + lax.axis_index("c")
        pltpu.sync_copy(lenp_hbm, lpv)
        pltpu.sync_copy(lenh_hbm, lhv)
        iota = lax.iota(jnp.int32, LANES)
        fzero = jnp.zeros((LANES,), jnp.float32)
        scat_base = iota * ACCW

        @pl.loop(0, NTCH * ACCW, step=4)
        def _(i):
            for q in range(4):
                stage[pl.ds((i + q) * LANES, LANES)] = fzero

        lpvec = lpv[...]
        lhvec = lhv[...]

        def lane(vec, s):
            return lax.reduce_max(jnp.where(iota == s, vec, 0), axes=(0,))

        def slot_info(d):
            p = lax.rem(wid + NW - d, NW)
            pm = lax.rem(p, B)
            length = jnp.where(p < B, lane(lpvec, pm), lane(lhvec, pm))
            vcols = jnp.clip(length - (KCOLS + d * TCR), 0, TCR)
            return p, pm, vcols

        def start_dma(d, buf, sem):
            p, pm, vcols = slot_info(d)
            c0 = KCOLS + d * TCR

            @pl.when((vcols > 0) & (p < B))
            def _():
                pltpu.async_copy(
                    prem_hbm.at[:, pm, pl.ds(c0, TCR)],
                    buf.at[pl.ds(0, D), :], sem)

            @pl.when((vcols > 0) & (p >= B))
            def _():
                pltpu.async_copy(
                    hyp_hbm.at[:, pm, pl.ds(c0, TCR)],
                    buf.at[pl.ds(0, D), :], sem)

        def consume(d, buf, sem):
            p, pm, vcols = slot_info(d)
            c0 = KCOLS + d * TCR
            sbase = scat_base + d * (ACCW * LANES)

            @pl.when((vcols > 0) & (p < B))
            def _():
                pltpu.make_async_copy(
                    prem_hbm.at[:, pm, pl.ds(c0, TCR)],
                    buf.at[pl.ds(0, D), :], sem).wait()

            @pl.when((vcols > 0) & (p >= B))
            def _():
                pltpu.make_async_copy(
                    hyp_hbm.at[:, pm, pl.ds(c0, TCR)],
                    buf.at[pl.ds(0, D), :], sem).wait()

            @pl.when(vcols >= TCR)
            def _():
                @pl.loop(0, DPAD, step=2)
                def _(f0):
                    for f in (f0, f0 + 1):
                        x = [buf[f, pl.ds(c * LANES, LANES)]
                             for c in range(CVR)]
                        v = ((x[0] + x[1]) + (x[2] + x[3])) + \
                            ((x[4] + x[5]) + (x[6] + x[7]))
                        plsc.store_scatter(stage, [sbase + f], v)

            @pl.when((vcols > 0) & (vcols < TCR))
            def _():
                masks = [(c * LANES + iota) < vcols for c in range(CVR)]

                @pl.loop(0, DPAD, step=2)
                def _(f0):
                    for f in (f0, f0 + 1):
                        v = fzero
                        for c in range(CVR):
                            x = buf[f, pl.ds(c * LANES, LANES)]
                            v = v + jnp.where(masks[c], x, 0.0)
                        plsc.store_scatter(stage, [sbase + f], v)

        bufs = (buf0, buf1)
        sems = (sem0, sem1)
        start_dma(0, bufs[0], sems[0])
        for d in range(NTCH):
            if d + 1 < NTCH:
                start_dma(d + 1, bufs[(d + 1) % 2], sems[(d + 1) % 2])
            consume(d, bufs[d % 2], sems[d % 2])

        pltpu.sync_copy(stage, out_hbm.at[wid])

    out = k(prem_t, lens_p, hyp_t, lens_h)
    return out


def _finale(tcp, tch, tails, lengths_p, lengths_h):
    def body(tcp_ref, tch_ref, tail_ref, lp_ref, lh_ref, out_ref):
        tf = tail_ref[...]
        t = None
        for d in range(NTCH):
            x = tf[:, d, :]
            rolled = jnp.concatenate([x[d:NW], x[0:d]], axis=0) if d else x
            t = rolled if t is None else t + rolled
        tsum = t[:, 0:ACCW]
        for lane in range(1, LANES):
            tsum = tsum + t[:, lane * ACCW:(lane + 1) * ACCW]
        t300 = tsum[:, 0:D]
        p = (tcp_ref[...].T + t300[0:B, :]) / lp_ref[...]
        h = (tch_ref[...].T + t300[B:2 * B, :]) / lh_ref[...]
        out_ref[:, 0, :] = p
        out_ref[:, 1, :] = h
        out_ref[:, 2, :] = jnp.abs(p - h)
        out_ref[:, 3, :] = p * h

    out = pl.pallas_call(
        body,
        out_shape=jax.ShapeDtypeStruct((B, 4, D), jnp.float32),
    )(tcp, tch, tails.reshape(NW, NTCH, ACCW * LANES),
      lengths_p.astype(jnp.float32).reshape(B, 1),
      lengths_h.astype(jnp.float32).reshape(B, 1))
    return out.reshape(B, 4 * D)


def kernel(premises, lengths_premises, hypothesis, lengths_hypothesis):
    prem_t = jnp.transpose(premises, (2, 0, 1))
    hyp_t = jnp.transpose(hypothesis, (2, 0, 1))
    lp = lengths_premises.astype(jnp.int32)
    lh = lengths_hypothesis.astype(jnp.int32)
    tails = _sc_tail_sums(prem_t, lp, hyp_t, lh)
    tcp, tch = _tc_dense_sums(prem_t, hyp_t,
                              lp.reshape(1, B), lh.reshape(1, B))
    return _finale(tcp, tch, tails, lengths_premises, lengths_hypothesis)


# speedup vs baseline: 1.1751x; 1.1751x over previous
import dataclasses
import functools

import jax
import jax.numpy as jnp
from jax import lax
from jax.experimental import pallas as pl
from jax.experimental.pallas import tpu as pltpu
from jax.experimental.pallas import tpu_sc as plsc

B, L, D = 16, 2048, 300
NC, NS = 2, 16
NW = NC * NS
LANES = 16
NFG = (D + LANES - 1) // LANES
DPAD = NFG * LANES
PAIRS = 2 * B

KCOLS = 1280
CC = 128
TCR = 128
NTCH = (L - KCOLS) // TCR
CVR = TCR // LANES
ACCW = DPAD + 1


def _tc_dense_sums(prem_t, hyp_t, lens_p2, lens_h2):
    def body(lp_ref, lh_ref, p_ref, h_ref, pout_ref, hout_ref):
        j = pl.program_id(0)
        ci = lax.broadcasted_iota(jnp.int32, (1, B, CC), 2) + j * CC
        mp = (ci < lp_ref[...].reshape(1, B, 1)).astype(jnp.float32)
        mh = (ci < lh_ref[...].reshape(1, B, 1)).astype(jnp.float32)
        ps = jnp.sum(p_ref[...] * mp, axis=2)
        hs = jnp.sum(h_ref[...] * mh, axis=2)

        @pl.when(j == 0)
        def _():
            pout_ref[...] = ps
            hout_ref[...] = hs

        @pl.when(j > 0)
        def _():
            pout_ref[...] += ps
            hout_ref[...] += hs

    return pl.pallas_call(
        body,
        grid=(KCOLS // CC,),
        in_specs=[
            pl.BlockSpec((1, B), lambda j: (0, 0)),
            pl.BlockSpec((1, B), lambda j: (0, 0)),
            pl.BlockSpec((D, B, CC), lambda j: (0, 0, j)),
            pl.BlockSpec((D, B, CC), lambda j: (0, 0, j)),
        ],
        out_specs=[
            pl.BlockSpec((D, B), lambda j: (0, 0)),
            pl.BlockSpec((D, B), lambda j: (0, 0)),
        ],
        out_shape=[jax.ShapeDtypeStruct((D, B), jnp.float32)] * 2,
    )(lens_p2, lens_h2, prem_t, hyp_t)


def _sc_tail_sums(prem_t, lens_p, hyp_t, lens_h):
    mesh = plsc.VectorSubcoreMesh(
        core_axis_name="c", subcore_axis_name="s",
        num_cores=NC, num_subcores=NS)
    cp = pltpu.CompilerParams()
    if "needs_layout_passes" in pltpu.CompilerParams.__dataclass_fields__:
        cp = dataclasses.replace(cp, needs_layout_passes=False)

    @functools.partial(
        pl.kernel,
        compiler_params=cp,
        out_type=jax.ShapeDtypeStruct((NW, NTCH * ACCW * LANES),
                                      jnp.float32),
        mesh=mesh,
        scratch_types=[
            pltpu.VMEM((DPAD, TCR), jnp.float32),
            pltpu.VMEM((DPAD, TCR), jnp.float32),
            pltpu.VMEM((NTCH * ACCW * LANES,), jnp.float32),
            pltpu.VMEM((LANES,), jnp.int32),
            pltpu.VMEM((LANES,), jnp.int32),
            pltpu.SemaphoreType.DMA,
            pltpu.SemaphoreType.DMA,
        ],
    )
    def k(prem_hbm, lenp_hbm, hyp_hbm, lenh_hbm, out_hbm,
          buf0, buf1, stage, lpv, lhv, sem0, sem1):
        wid = lax.axis_index("s") * NC + lax.axis_index("c")
        pltpu.sync_copy(lenp_hbm, lpv)
        pltpu.sync_copy(lenh_hbm, lhv)
        iota = lax.iota(jnp.int32, LANES)
        fzero = jnp.zeros((LANES,), jnp.float32)
        scat_base = iota * ACCW

        @pl.loop(0, NTCH * ACCW, step=4)
        def _(i):
            for q in range(4):
                stage[pl.ds((i + q) * LANES, LANES)] = fzero

        lpvec = lpv[...]
        lhvec = lhv[...]

        def lane(vec, s):
            return lax.reduce_max(jnp.where(iota == s, vec, 0), axes=(0,))

        def slot_info(d):
            p = lax.rem(wid + NW - d, NW)
            pm = lax.rem(p, B)
            length = jnp.where(p < B, lane(lpvec, pm), lane(lhvec, pm))
            vcols = jnp.clip(length - (KCOLS + d * TCR), 0, TCR)
            return p, pm, vcols

        def start_dma(d, buf, sem):
            p, pm, vcols = slot_info(d)
            c0 = KCOLS + d * TCR

            @pl.when((vcols > 0) & (p < B))
            def _():
                pltpu.async_copy(
                    prem_hbm.at[:, pm, pl.ds(c0, TCR)],
                    buf.at[pl.ds(0, D), :], sem)

            @pl.when((vcols > 0) & (p >= B))
            def _():
                pltpu.async_copy(
                    hyp_hbm.at[:, pm, pl.ds(c0, TCR)],
                    buf.at[pl.ds(0, D), :], sem)

        def consume(d, buf, sem):
            p, pm, vcols = slot_info(d)
            c0 = KCOLS + d * TCR
            sbase = scat_base + d * (ACCW * LANES)

            @pl.when((vcols > 0) & (p < B))
            def _():
                pltpu.make_async_copy(
                    prem_hbm.at[:, pm, pl.ds(c0, TCR)],
                    buf.at[pl.ds(0, D), :], sem).wait()

            @pl.when((vcols > 0) & (p >= B))
            def _():
                pltpu.make_async_copy(
                    hyp_hbm.at[:, pm, pl.ds(c0, TCR)],
                    buf.at[pl.ds(0, D), :], sem).wait()

            @pl.when(vcols >= TCR)
            def _():
                @pl.loop(0, DPAD, step=2)
                def _(f0):
                    for f in (f0, f0 + 1):
                        x = [buf[f, pl.ds(c * LANES, LANES)]
                             for c in range(CVR)]
                        v = ((x[0] + x[1]) + (x[2] + x[3])) + \
                            ((x[4] + x[5]) + (x[6] + x[7]))
                        plsc.store_scatter(stage, [sbase + f], v)

            @pl.when((vcols > 0) & (vcols < TCR))
            def _():
                masks = [(c * LANES + iota) < vcols for c in range(CVR)]

                @pl.loop(0, DPAD, step=2)
                def _(f0):
                    for f in (f0, f0 + 1):
                        v = fzero
                        for c in range(CVR):
                            x = buf[f, pl.ds(c * LANES, LANES)]
                            v = v + jnp.where(masks[c], x, 0.0)
                        plsc.store_scatter(stage, [sbase + f], v)

        bufs = (buf0, buf1)
        sems = (sem0, sem1)
        start_dma(0, bufs[0], sems[0])
        for d in range(NTCH):
            if d + 1 < NTCH:
                start_dma(d + 1, bufs[(d + 1) % 2], sems[(d + 1) % 2])
            consume(d, bufs[d % 2], sems[d % 2])

        pltpu.sync_copy(stage, out_hbm.at[wid])

    out = k(prem_t, lens_p, hyp_t, lens_h)
    return out


def _finale(tcp, tch, tails, lengths_p, lengths_h):
    def body(tcp_ref, tch_ref, tail_ref, lp_ref, lh_ref, out_ref):
        tf = tail_ref[...]
        sl = ACCW * LANES
        t = None
        for d in range(NTCH):
            x = tf[:, d * sl:(d + 1) * sl]
            rolled = jnp.concatenate([x[d:NW], x[0:d]], axis=0) if d else x
            t = rolled if t is None else t + rolled
        tsum = t[:, 0:ACCW]
        for lane in range(1, LANES):
            tsum = tsum + t[:, lane * ACCW:(lane + 1) * ACCW]
        t300 = tsum[:, 0:D]
        p = (tcp_ref[...].T + t300[0:B, :]) / lp_ref[...]
        h = (tch_ref[...].T + t300[B:2 * B, :]) / lh_ref[...]
        out_ref[:, 0, :] = p
        out_ref[:, 1, :] = h
        out_ref[:, 2, :] = jnp.abs(p - h)
        out_ref[:, 3, :] = p * h

    out = pl.pallas_call(
        body,
        out_shape=jax.ShapeDtypeStruct((B, 4, D), jnp.float32),
    )(tcp, tch, tails,
      lengths_p.astype(jnp.float32).reshape(B, 1),
      lengths_h.astype(jnp.float32).reshape(B, 1))
    return out.reshape(B, 4 * D)


def kernel(premises, lengths_premises, hypothesis, lengths_hypothesis):
    prem_t = jnp.transpose(premises, (2, 0, 1))
    hyp_t = jnp.transpose(hypothesis, (2, 0, 1))
    lp = lengths_premises.astype(jnp.int32)
    lh = lengths_hypothesis.astype(jnp.int32)
    tails = _sc_tail_sums(prem_t, lp, hyp_t, lh)
    tcp, tch = _tc_dense_sums(prem_t, hyp_t,
                              lp.reshape(1, B), lh.reshape(1, B))
    return _finale(tcp, tch, tails, lengths_premises, lengths_hypothesis)


# speedup vs baseline: 1.2407x; 1.0559x over previous
import dataclasses
import functools

import jax
import jax.numpy as jnp
from jax import lax
from jax.experimental import pallas as pl
from jax.experimental.pallas import tpu as pltpu
from jax.experimental.pallas import tpu_sc as plsc

B, L, D = 16, 2048, 300
NC, NS = 2, 16
NW = NC * NS
LANES = 16
NFG = (D + LANES - 1) // LANES
DPAD = NFG * LANES
PAIRS = 2 * B

KCOLS = 1280
CC = 128
TCR = 128
NTCH = (L - KCOLS) // TCR
CVR = TCR // LANES
SLOT = DPAD


def _tc_dense_sums(prem_t, hyp_t, lens_p2, lens_h2):
    def body(lp_ref, lh_ref, p_ref, h_ref, pout_ref, hout_ref):
        j = pl.program_id(0)
        ci = lax.broadcasted_iota(jnp.int32, (1, B, CC), 2) + j * CC
        mp = (ci < lp_ref[...].reshape(1, B, 1)).astype(jnp.float32)
        mh = (ci < lh_ref[...].reshape(1, B, 1)).astype(jnp.float32)
        ps = jnp.sum(p_ref[...] * mp, axis=2)
        hs = jnp.sum(h_ref[...] * mh, axis=2)

        @pl.when(j == 0)
        def _():
            pout_ref[...] = ps
            hout_ref[...] = hs

        @pl.when(j > 0)
        def _():
            pout_ref[...] += ps
            hout_ref[...] += hs

    return pl.pallas_call(
        body,
        grid=(KCOLS // CC,),
        in_specs=[
            pl.BlockSpec((1, B), lambda j: (0, 0)),
            pl.BlockSpec((1, B), lambda j: (0, 0)),
            pl.BlockSpec((D, B, CC), lambda j: (0, 0, j)),
            pl.BlockSpec((D, B, CC), lambda j: (0, 0, j)),
        ],
        out_specs=[
            pl.BlockSpec((D, B), lambda j: (0, 0)),
            pl.BlockSpec((D, B), lambda j: (0, 0)),
        ],
        out_shape=[jax.ShapeDtypeStruct((D, B), jnp.float32)] * 2,
    )(lens_p2, lens_h2, prem_t, hyp_t)


def _sc_tail_sums(prem_t, lens_p, hyp_t, lens_h):
    mesh = plsc.VectorSubcoreMesh(
        core_axis_name="c", subcore_axis_name="s",
        num_cores=NC, num_subcores=NS)
    cp = pltpu.CompilerParams()
    if "needs_layout_passes" in pltpu.CompilerParams.__dataclass_fields__:
        cp = dataclasses.replace(cp, needs_layout_passes=False)

    @functools.partial(
        pl.kernel,
        compiler_params=cp,
        out_type=jax.ShapeDtypeStruct((NW, NTCH * SLOT), jnp.float32),
        mesh=mesh,
        scratch_types=[
            pltpu.VMEM((DPAD, TCR), jnp.float32),
            pltpu.VMEM((DPAD, TCR), jnp.float32),
            pltpu.VMEM((DPAD, TCR), jnp.float32),
            pltpu.VMEM((NTCH * SLOT,), jnp.float32),
            pltpu.VMEM((LANES,), jnp.int32),
            pltpu.VMEM((LANES,), jnp.int32),
            pltpu.SemaphoreType.DMA,
            pltpu.SemaphoreType.DMA,
            pltpu.SemaphoreType.DMA,
        ],
    )
    def k(prem_hbm, lenp_hbm, hyp_hbm, lenh_hbm, out_hbm,
          buf0, buf1, buf2, stage, lpv, lhv, sem0, sem1, sem2):
        wid = lax.axis_index("s") * NC + lax.axis_index("c")
        pltpu.sync_copy(lenp_hbm, lpv)
        pltpu.sync_copy(lenh_hbm, lhv)
        iota = lax.iota(jnp.int32, LANES)
        fzero = jnp.zeros((LANES,), jnp.float32)
        shuf = [iota ^ (1 << kk) for kk in range(4)]
        gdims = lax.GatherDimensionNumbers(
            offset_dims=(), collapsed_slice_dims=(0,), start_index_map=(0,))

        def lane_shuffle(v, m):
            return lax.gather(
                v, m[:, None], gdims, slice_sizes=(1,),
                mode=lax.GatherScatterMode.PROMISE_IN_BOUNDS)

        def fold_lanes(v):
            for m in shuf:
                v = v + lane_shuffle(v, m)
            return v

        @pl.loop(0, NTCH * SLOT // LANES, step=4)
        def _(i):
            for q in range(4):
                stage[pl.ds((i + q) * LANES, LANES)] = fzero

        lpvec = lpv[...]
        lhvec = lhv[...]

        def lane(vec, s):
            return lax.reduce_max(jnp.where(iota == s, vec, 0), axes=(0,))

        def slot_info(d):
            p = lax.rem(wid + NW - d, NW)
            pm = lax.rem(p, B)
            length = jnp.where(p < B, lane(lpvec, pm), lane(lhvec, pm))
            vcols = jnp.clip(length - (KCOLS + d * TCR), 0, TCR)
            return p, pm, vcols

        def start_dma(d, buf, sem):
            p, pm, vcols = slot_info(d)
            c0 = KCOLS + d * TCR

            @pl.when((vcols > 0) & (p < B))
            def _():
                pltpu.async_copy(
                    prem_hbm.at[:, pm, pl.ds(c0, TCR)],
                    buf.at[pl.ds(0, D), :], sem)

            @pl.when((vcols > 0) & (p >= B))
            def _():
                pltpu.async_copy(
                    hyp_hbm.at[:, pm, pl.ds(c0, TCR)],
                    buf.at[pl.ds(0, D), :], sem)

        def consume(d, buf, sem):
            p, pm, vcols = slot_info(d)
            c0 = KCOLS + d * TCR

            @pl.when((vcols > 0) & (p < B))
            def _():
                pltpu.make_async_copy(
                    prem_hbm.at[:, pm, pl.ds(c0, TCR)],
                    buf.at[pl.ds(0, D), :], sem).wait()

            @pl.when((vcols > 0) & (p >= B))
            def _():
                pltpu.make_async_copy(
                    hyp_hbm.at[:, pm, pl.ds(c0, TCR)],
                    buf.at[pl.ds(0, D), :], sem).wait()

            def feature_sum(f, masks):
                x = [buf[f, pl.ds(c * LANES, LANES)] for c in range(CVR)]
                if masks is None:
                    return ((x[0] + x[1]) + (x[2] + x[3])) + \
                           ((x[4] + x[5]) + (x[6] + x[7]))
                v = fzero
                for c in range(CVR):
                    v = v + jnp.where(masks[c], x[c], 0.0)
                return v

            def fg_loop(masks):
                @pl.loop(0, NFG)
                def _(fg):
                    f0 = fg * LANES
                    w = fzero
                    for j in range(LANES):
                        v = fold_lanes(feature_sum(f0 + j, masks))
                        w = jnp.where(iota == j, v, w)
                    stage[pl.ds(d * SLOT + f0, LANES)] = w

            @pl.when(vcols >= TCR)
            def _():
                fg_loop(None)

            @pl.when((vcols > 0) & (vcols < TCR))
            def _():
                fg_loop([(c * LANES + iota) < vcols for c in range(CVR)])

        bufs = (buf0, buf1, buf2)
        sems = (sem0, sem1, sem2)
        start_dma(0, bufs[0], sems[0])
        start_dma(1, bufs[1], sems[1])
        for d in range(NTCH):
            if d + 2 < NTCH:
                start_dma(d + 2, bufs[(d + 2) % 3], sems[(d + 2) % 3])
            consume(d, bufs[d % 3], sems[d % 3])

        pltpu.sync_copy(stage, out_hbm.at[wid])

    out = k(prem_t, lens_p, hyp_t, lens_h)
    return out


def _finale(tcp, tch, tails, lengths_p, lengths_h):
    def body(tcp_ref, tch_ref, tail_ref, lp_ref, lh_ref, out_ref):
        tf = tail_ref[...]
        t = None
        for d in range(NTCH):
            x = tf[:, d * SLOT:(d + 1) * SLOT]
            rolled = jnp.concatenate([x[d:NW], x[0:d]], axis=0) if d else x
            t = rolled if t is None else t + rolled
        t300 = t[:, 0:D]
        p = (tcp_ref[...].T + t300[0:B, :]) / lp_ref[...]
        h = (tch_ref[...].T + t300[B:2 * B, :]) / lh_ref[...]
        out_ref[:, 0, :] = p
        out_ref[:, 1, :] = h
        out_ref[:, 2, :] = jnp.abs(p - h)
        out_ref[:, 3, :] = p * h

    out = pl.pallas_call(
        body,
        out_shape=jax.ShapeDtypeStruct((B, 4, D), jnp.float32),
    )(tcp, tch, tails,
      lengths_p.astype(jnp.float32).reshape(B, 1),
      lengths_h.astype(jnp.float32).reshape(B, 1))
    return out.reshape(B, 4 * D)


def kernel(premises, lengths_premises, hypothesis, lengths_hypothesis):
    prem_t = jnp.transpose(premises, (2, 0, 1))
    hyp_t = jnp.transpose(hypothesis, (2, 0, 1))
    lp = lengths_premises.astype(jnp.int32)
    lh = lengths_hypothesis.astype(jnp.int32)
    tails = _sc_tail_sums(prem_t, lp, hyp_t, lh)
    tcp, tch = _tc_dense_sums(prem_t, hyp_t,
                              lp.reshape(1, B), lh.reshape(1, B))
    return _finale(tcp, tch, tails, lengths_premises, lengths_hypothesis)
